# Initial kernel scaffold; baseline (speedup 1.0000x reference)
import jax
import jax.numpy as jnp
from jax.experimental import pallas as pl


def kernel(x, gate_w, W1, b1, W2, b2, W3, b3, Ws1, bs1, Ws2, bs2, Ws3, bs3):
    raise NotImplementedError("write your pallas kernel here")


# speedup vs baseline: 1.9438x; 1.9438x over previous
import functools

import jax
import jax.numpy as jnp
from jax import lax
from jax.experimental import pallas as pl
from jax.experimental.pallas import tpu as pltpu
from jax.experimental.pallas import tpu_sc as plsc

N = 2048
D = 1024
DFF = 2048
E = 8
TOPK = 2
T = 256
NT = 24
R = NT * T
NTPAD = 128
PAIRS = N * TOPK

NC = 2
NS = 16
NW = NC * NS
TOK_W = N // NW
CH = 32

_f32 = jnp.float32
_i32 = jnp.int32


def _router_body(x_ref, gw_ref, pos_ref, ewp_ref, te_ref, va_ref, b_scr, exc_scr):
    xv = x_ref[...]
    gw = gw_ref[...]
    logits = lax.dot_general(gw, xv, (((1,), (1,)), ((), ())),
                             preferred_element_type=_f32)
    m = jnp.max(logits, axis=0, keepdims=True)
    p = jnp.exp(logits - m)
    scores = p / jnp.sum(p, axis=0, keepdims=True)

    e_ids = lax.broadcasted_iota(_i32, (E, N), 0)
    s0 = jnp.max(scores, axis=0, keepdims=True)
    i0 = jnp.min(jnp.where(scores == s0, e_ids, E), axis=0, keepdims=True)
    mask0 = e_ids == i0
    scores1 = jnp.where(mask0, -jnp.inf, scores)
    s1 = jnp.max(scores1, axis=0, keepdims=True)
    i1 = jnp.min(jnp.where(scores1 == s1, e_ids, E), axis=0, keepdims=True)
    mask1 = e_ids == i1

    r = jnp.exp(s1 - s0)
    ew0 = 1.0 / (1.0 + r)
    ew1 = r / (1.0 + r)

    b_scr[...] = mask0.astype(_f32) + mask1.astype(_f32)

    cb = 256
    iu = lax.broadcasted_iota(_i32, (cb, cb), 0)
    ju = lax.broadcasted_iota(_i32, (cb, cb), 1)
    upper = (iu < ju).astype(_f32)

    def chunk(c, tot):
        bc = b_scr[:, pl.ds(c * cb, cb)]
        exc = lax.dot_general(bc, upper, (((1,), (0,)), ((), ())),
                              preferred_element_type=_f32) + tot
        exc_scr[:, pl.ds(c * cb, cb)] = exc
        return tot + jnp.sum(bc, axis=1, keepdims=True)

    counts = lax.fori_loop(0, N // cb, chunk, jnp.zeros((E, 1), _f32))

    pc = jnp.ceil(counts / float(T)) * float(T)
    il = lax.broadcasted_iota(_i32, (E, E), 0)
    jl = lax.broadcasted_iota(_i32, (E, E), 1)
    lower = (il > jl).astype(_f32)
    ps = lax.dot_general(lower, pc, (((1,), (0,)), ((), ())),
                         preferred_element_type=_f32)

    posall = exc_scr[...] + ps
    pos0 = jnp.sum(jnp.where(mask0, posall, 0.0), axis=0, keepdims=True)
    pos1 = jnp.sum(jnp.where(mask1, posall, 0.0), axis=0, keepdims=True)
    pos_ref[0:1, :] = pos0.astype(_i32)
    pos_ref[1:2, :] = pos1.astype(_i32)
    ewp_ref[0:1, :] = ew0
    ewp_ref[1:2, :] = ew1

    tt = (lax.broadcasted_iota(_i32, (1, NTPAD), 1) * T).astype(_f32)
    ge = (tt >= ps).astype(_i32)
    te = jnp.sum(ge, axis=0, keepdims=True) - 1
    te = jnp.clip(te, 0, E - 1)
    inseg = jnp.logical_and(tt >= ps, tt < ps + counts)
    va = jnp.sum(inseg.astype(_i32), axis=0, keepdims=True)
    te_ref[...] = te
    va_ref[...] = va


def _router(x2, gate_w):
    return pl.pallas_call(
        _router_body,
        out_shape=(
            jax.ShapeDtypeStruct((TOPK, N), _i32),
            jax.ShapeDtypeStruct((TOPK, N), _f32),
            jax.ShapeDtypeStruct((1, NTPAD), _i32),
            jax.ShapeDtypeStruct((1, NTPAD), _i32),
        ),
        scratch_shapes=[
            pltpu.VMEM((E, N), _f32),
            pltpu.VMEM((E, N), _f32),
        ],
    )(x2, gate_w)


def _sc_mesh():
    return plsc.VectorSubcoreMesh(core_axis_name="c", subcore_axis_name="s",
                                  num_cores=NC, num_subcores=NS)


def _dispatch(x2, pos, ewp):
    return pl.kernel(
        _dispatch_body,
        out_type=(
            jax.ShapeDtypeStruct((R, D), _f32),
            jax.ShapeDtypeStruct((R,), _f32),
        ),
        mesh=_sc_mesh(),
        scratch_types=[
            pltpu.VMEM((TOK_W,), _i32),
            pltpu.VMEM((TOK_W,), _i32),
            pltpu.VMEM((TOK_W,), _f32),
            pltpu.VMEM((TOK_W,), _f32),
            pltpu.VMEM((TOK_W, D), _f32),
            pltpu.SemaphoreType.DMA,
            pltpu.SemaphoreType.DMA,
            pltpu.SemaphoreType.DMA,
            pltpu.SemaphoreType.DMA,
        ],
    )(x2, pos, ewp)


def _dispatch_body(x_hbm, pos_hbm, ewp_hbm, xs_hbm, ews_hbm,
                   idx0_v, idx1_v, ew0_v, ew1_v, rows_v, sem0, sem1, sem2, sem3):
    wid = lax.axis_index("s") * NC + lax.axis_index("c")
    base = wid * TOK_W
    pltpu.sync_copy(x_hbm.at[pl.ds(base, TOK_W)], rows_v)
    pltpu.sync_copy(pos_hbm.at[0, pl.ds(base, TOK_W)], idx0_v)
    pltpu.sync_copy(pos_hbm.at[1, pl.ds(base, TOK_W)], idx1_v)
    pltpu.sync_copy(ewp_hbm.at[0, pl.ds(base, TOK_W)], ew0_v)
    pltpu.sync_copy(ewp_hbm.at[1, pl.ds(base, TOK_W)], ew1_v)
    cp0 = pltpu.async_copy(rows_v, xs_hbm.at[idx0_v], sem0)
    cp1 = pltpu.async_copy(rows_v, xs_hbm.at[idx1_v], sem1)
    cp2 = pltpu.async_copy(ew0_v, ews_hbm.at[idx0_v], sem2)
    cp3 = pltpu.async_copy(ew1_v, ews_hbm.at[idx1_v], sem3)
    cp0.wait()
    cp1.wait()
    cp2.wait()
    cp3.wait()


def _gffn_body(te_ref, va_ref, xs_ref, w1_ref, b1_ref, w3_ref, b3_ref,
               w2_ref, b2_ref, ews_ref, out_ref):
    t = pl.program_id(0)

    @pl.when(va_ref[t] != 0)
    def _():
        xb = xs_ref[...]
        h1 = lax.dot_general(xb, w1_ref[0], (((1,), (1,)), ((), ())),
                             preferred_element_type=_f32) + b1_ref[0]
        h3 = lax.dot_general(xb, w3_ref[0], (((1,), (1,)), ((), ())),
                             preferred_element_type=_f32) + b3_ref[0]
        z = h1 * h3
        h = z / (1.0 + jnp.exp(-z))
        acc = lax.dot_general(h, w2_ref[0], (((1,), (1,)), ((), ())),
                              preferred_element_type=_f32) + b2_ref[0]
        out_ref[...] = acc * ews_ref[...]


def _grouped_ffn(te, va, xs, W1, b1, W3, b3, W2, b2, ews2d):
    grid_spec = pltpu.PrefetchScalarGridSpec(
        num_scalar_prefetch=2,
        grid=(NT,),
        in_specs=[
            pl.BlockSpec((T, D), lambda t, te, va: (t, 0)),
            pl.BlockSpec((1, DFF, D), lambda t, te, va: (te[t], 0, 0)),
            pl.BlockSpec((1, 1, DFF), lambda t, te, va: (te[t], 0, 0)),
            pl.BlockSpec((1, DFF, D), lambda t, te, va: (te[t], 0, 0)),
            pl.BlockSpec((1, 1, DFF), lambda t, te, va: (te[t], 0, 0)),
            pl.BlockSpec((1, D, DFF), lambda t, te, va: (te[t], 0, 0)),
            pl.BlockSpec((1, 1, D), lambda t, te, va: (te[t], 0, 0)),
            pl.BlockSpec((T, 1), lambda t, te, va: (t, 0)),
        ],
        out_specs=pl.BlockSpec((T, D), lambda t, te, va: (t, 0)),
    )
    return pl.pallas_call(
        _gffn_body,
        grid_spec=grid_spec,
        out_shape=jax.ShapeDtypeStruct((R, D), _f32),
    )(te, va, xs, W1, b1, W3, b3, W2, b2, ews2d)


def _shared_body(x_ref, w1_ref, b1_ref, w3_ref, b3_ref, w2_ref, b2_ref, out_ref):
    xb = x_ref[...]
    h1 = lax.dot_general(xb, w1_ref[...], (((1,), (1,)), ((), ())),
                         preferred_element_type=_f32) + b1_ref[...]
    h3 = lax.dot_general(xb, w3_ref[...], (((1,), (1,)), ((), ())),
                         preferred_element_type=_f32) + b3_ref[...]
    z = h1 * h3
    h = z / (1.0 + jnp.exp(-z))
    out_ref[...] = lax.dot_general(h, w2_ref[...], (((1,), (1,)), ((), ())),
                                   preferred_element_type=_f32) + b2_ref[...]


def _shared_ffn(x2, Ws1, bs1, Ws3, bs3, Ws2, bs2):
    nst = N // T
    return pl.pallas_call(
        _shared_body,
        grid=(nst,),
        in_specs=[
            pl.BlockSpec((T, D), lambda t: (t, 0)),
            pl.BlockSpec((DFF, D), lambda t: (0, 0)),
            pl.BlockSpec((1, DFF), lambda t: (0, 0)),
            pl.BlockSpec((DFF, D), lambda t: (0, 0)),
            pl.BlockSpec((1, DFF), lambda t: (0, 0)),
            pl.BlockSpec((D, DFF), lambda t: (0, 0)),
            pl.BlockSpec((1, D), lambda t: (0, 0)),
        ],
        out_specs=pl.BlockSpec((T, D), lambda t: (t, 0)),
        out_shape=jax.ShapeDtypeStruct((N, D), _f32),
    )(x2, Ws1, bs1, Ws3, bs3, Ws2, bs2)


def _combine(sh, ys, pos):
    return pl.kernel(
        _combine_body,
        out_type=jax.ShapeDtypeStruct((N, D), _f32),
        mesh=_sc_mesh(),
        scratch_types=[
            pltpu.VMEM((CH,), _i32),
            pltpu.VMEM((CH,), _i32),
            pltpu.VMEM((CH, D), _f32),
            pltpu.VMEM((CH, D), _f32),
            pltpu.VMEM((CH, D), _f32),
            pltpu.SemaphoreType.DMA,
            pltpu.SemaphoreType.DMA,
            pltpu.SemaphoreType.DMA,
        ],
    )(sh, ys, pos)


def _combine_body(sh_hbm, ys_hbm, pos_hbm, y_hbm,
                  idx0_v, idx1_v, g0_v, g1_v, shb_v, sem0, sem1, sem2):
    wid = lax.axis_index("s") * NC + lax.axis_index("c")
    base = wid * TOK_W
    for c in range(TOK_W // CH):
        b = base + c * CH
        pltpu.sync_copy(pos_hbm.at[0, pl.ds(b, CH)], idx0_v)
        pltpu.sync_copy(pos_hbm.at[1, pl.ds(b, CH)], idx1_v)
        a0 = pltpu.async_copy(ys_hbm.at[idx0_v], g0_v, sem0)
        a1 = pltpu.async_copy(ys_hbm.at[idx1_v], g1_v, sem1)
        a2 = pltpu.async_copy(sh_hbm.at[pl.ds(b, CH)], shb_v, sem2)
        a0.wait()
        a1.wait()
        a2.wait()

        def body(i, carry):
            j = i // (D // 16)
            k = (i % (D // 16)) * 16
            shb_v[j, pl.ds(k, 16)] = (shb_v[j, pl.ds(k, 16)]
                                      + g0_v[j, pl.ds(k, 16)]
                                      + g1_v[j, pl.ds(k, 16)])
            return carry

        lax.fori_loop(0, CH * (D // 16), body, 0)
        pltpu.sync_copy(shb_v, y_hbm.at[pl.ds(b, CH)])


def kernel(x, gate_w, W1, b1, W2, b2, W3, b3, Ws1, bs1, Ws2, bs2, Ws3, bs3):
    shape = x.shape
    x2 = x.reshape(-1, shape[-1])

    pos, ewp, te2d, va2d = _router(x2, gate_w)
    te = te2d[0]
    va = va2d[0]

    xs, ews = _dispatch(x2, pos, ewp)
    ys = _grouped_ffn(te, va, xs, W1, b1.reshape(E, 1, DFF), W3,
                      b3.reshape(E, 1, DFF), W2, b2.reshape(E, 1, D),
                      ews.reshape(R, 1))
    sh = _shared_ffn(x2, Ws1, bs1.reshape(1, DFF), Ws3, bs3.reshape(1, DFF),
                     Ws2, bs2.reshape(1, D))
    y = _combine(sh, ys, pos)
    return y.reshape(shape)


# speedup vs baseline: 2.0052x; 1.0316x over previous
import functools

import jax
import jax.numpy as jnp
from jax import lax
from jax.experimental import pallas as pl
from jax.experimental.pallas import tpu as pltpu
from jax.experimental.pallas import tpu_sc as plsc

N = 2048
D = 1024
DFF = 2048
E = 8
TOPK = 2
T = 256
NT = 24
R = NT * T
NTPAD = 128
PAIRS = N * TOPK

NC = 2
NS = 16
NW = NC * NS
TOK_W = N // NW
CH = 32

_f32 = jnp.float32
_i32 = jnp.int32


def _router_body(x_ref, gw_ref, pos_ref, ewp_ref, te_ref, va_ref, ct_ref,
                 b_scr, exc_scr):
    xv = x_ref[...]
    gw = gw_ref[...]
    logits = lax.dot_general(gw, xv, (((1,), (1,)), ((), ())),
                             preferred_element_type=_f32)
    m = jnp.max(logits, axis=0, keepdims=True)
    p = jnp.exp(logits - m)
    scores = p / jnp.sum(p, axis=0, keepdims=True)

    e_ids = lax.broadcasted_iota(_i32, (E, N), 0)
    s0 = jnp.max(scores, axis=0, keepdims=True)
    i0 = jnp.min(jnp.where(scores == s0, e_ids, E), axis=0, keepdims=True)
    mask0 = e_ids == i0
    scores1 = jnp.where(mask0, -jnp.inf, scores)
    s1 = jnp.max(scores1, axis=0, keepdims=True)
    i1 = jnp.min(jnp.where(scores1 == s1, e_ids, E), axis=0, keepdims=True)
    mask1 = e_ids == i1

    r = jnp.exp(s1 - s0)
    ew0 = 1.0 / (1.0 + r)
    ew1 = r / (1.0 + r)

    b_scr[...] = mask0.astype(_f32) + mask1.astype(_f32)

    cb = 256
    iu = lax.broadcasted_iota(_i32, (cb, cb), 0)
    ju = lax.broadcasted_iota(_i32, (cb, cb), 1)
    upper = (iu < ju).astype(_f32)

    def chunk(c, tot):
        bc = b_scr[:, pl.ds(c * cb, cb)]
        exc = lax.dot_general(bc, upper, (((1,), (0,)), ((), ())),
                              preferred_element_type=_f32) + tot
        exc_scr[:, pl.ds(c * cb, cb)] = exc
        return tot + jnp.sum(bc, axis=1, keepdims=True)

    counts = lax.fori_loop(0, N // cb, chunk, jnp.zeros((E, 1), _f32))

    pc = jnp.ceil(counts / float(T)) * float(T)
    il = lax.broadcasted_iota(_i32, (E, E), 0)
    jl = lax.broadcasted_iota(_i32, (E, E), 1)
    lower = (il > jl).astype(_f32)
    ps = lax.dot_general(lower, pc, (((1,), (0,)), ((), ())),
                         preferred_element_type=_f32)

    posall = exc_scr[...] + ps
    pos0 = jnp.sum(jnp.where(mask0, posall, 0.0), axis=0, keepdims=True)
    pos1 = jnp.sum(jnp.where(mask1, posall, 0.0), axis=0, keepdims=True)
    pos_ref[0:1, :] = pos0.astype(_i32)
    pos_ref[1:2, :] = pos1.astype(_i32)
    ewp_ref[0:1, :] = ew0
    ewp_ref[1:2, :] = ew1

    tt = (lax.broadcasted_iota(_i32, (1, NTPAD), 1) * T).astype(_f32)
    ge = (tt >= ps).astype(_i32)
    te = jnp.sum(ge, axis=0, keepdims=True) - 1
    te = jnp.clip(te, 0, E - 1)
    inseg = jnp.logical_and(tt >= ps, tt < ps + counts)
    va = jnp.sum(inseg.astype(_i32), axis=0, keepdims=True)
    e_col = lax.broadcasted_iota(_i32, (E, 1), 0)
    last_e = jnp.max(jnp.where(counts > 0, e_col, -1), axis=0, keepdims=True)
    nvalid = jnp.sum(va, axis=1, keepdims=True)
    tids = lax.broadcasted_iota(_i32, (1, NTPAD), 1)
    te_ref[...] = jnp.where(va != 0, te, last_e)
    va_ref[...] = va
    ct_ref[...] = jnp.minimum(tids, nvalid - 1)


def _router(x2, gate_w):
    return pl.pallas_call(
        _router_body,
        out_shape=(
            jax.ShapeDtypeStruct((TOPK, N), _i32),
            jax.ShapeDtypeStruct((TOPK, N), _f32),
            jax.ShapeDtypeStruct((1, NTPAD), _i32),
            jax.ShapeDtypeStruct((1, NTPAD), _i32),
            jax.ShapeDtypeStruct((1, NTPAD), _i32),
        ),
        scratch_shapes=[
            pltpu.VMEM((E, N), _f32),
            pltpu.VMEM((E, N), _f32),
        ],
    )(x2, gate_w)


def _sc_mesh():
    return plsc.VectorSubcoreMesh(core_axis_name="c", subcore_axis_name="s",
                                  num_cores=NC, num_subcores=NS)


def _dispatch(x2, pos, ewp):
    return pl.kernel(
        _dispatch_body,
        out_type=(
            jax.ShapeDtypeStruct((R, D), _f32),
            jax.ShapeDtypeStruct((R,), _f32),
        ),
        mesh=_sc_mesh(),
        scratch_types=[
            pltpu.VMEM((TOK_W,), _i32),
            pltpu.VMEM((TOK_W,), _i32),
            pltpu.VMEM((TOK_W,), _f32),
            pltpu.VMEM((TOK_W,), _f32),
            pltpu.VMEM((TOK_W, D), _f32),
            pltpu.SemaphoreType.DMA,
            pltpu.SemaphoreType.DMA,
            pltpu.SemaphoreType.DMA,
            pltpu.SemaphoreType.DMA,
        ],
    )(x2, pos, ewp)


def _dispatch_body(x_hbm, pos_hbm, ewp_hbm, xs_hbm, ews_hbm,
                   idx0_v, idx1_v, ew0_v, ew1_v, rows_v, sem0, sem1, sem2, sem3):
    wid = lax.axis_index("s") * NC + lax.axis_index("c")
    base = wid * TOK_W
    pltpu.sync_copy(x_hbm.at[pl.ds(base, TOK_W)], rows_v)
    pltpu.sync_copy(pos_hbm.at[0, pl.ds(base, TOK_W)], idx0_v)
    pltpu.sync_copy(pos_hbm.at[1, pl.ds(base, TOK_W)], idx1_v)
    pltpu.sync_copy(ewp_hbm.at[0, pl.ds(base, TOK_W)], ew0_v)
    pltpu.sync_copy(ewp_hbm.at[1, pl.ds(base, TOK_W)], ew1_v)
    cp0 = pltpu.async_copy(rows_v, xs_hbm.at[idx0_v], sem0)
    cp1 = pltpu.async_copy(rows_v, xs_hbm.at[idx1_v], sem1)
    cp2 = pltpu.async_copy(ew0_v, ews_hbm.at[idx0_v], sem2)
    cp3 = pltpu.async_copy(ew1_v, ews_hbm.at[idx1_v], sem3)
    cp0.wait()
    cp1.wait()
    cp2.wait()
    cp3.wait()


def _gffn_inner(idx, xs_ref, w1_ref, b1_ref, w3_ref, b3_ref,
                w2_ref, b2_ref, ews_ref, out_ref, *, va_hbm):
    (t,) = idx

    @pl.when(va_hbm[0, t] != 0)
    def _():
        xb = xs_ref[...]
        h1 = lax.dot_general(xb, w1_ref[0], (((1,), (1,)), ((), ())),
                             preferred_element_type=_f32) + b1_ref[0]
        h3 = lax.dot_general(xb, w3_ref[0], (((1,), (1,)), ((), ())),
                             preferred_element_type=_f32) + b3_ref[0]
        z = h1 * h3
        h = z / (1.0 + jnp.exp(-z))
        acc = lax.dot_general(h, w2_ref[0], (((1,), (1,)), ((), ())),
                              preferred_element_type=_f32) + b2_ref[0]
        out_ref[...] = acc * ews_ref[...]


def _gffn_outer(te_ref, va_ref, ct_ref, xs_hbm, w1_hbm, b1_hbm, w3_hbm,
                b3_hbm, w2_hbm, b2_hbm, ews_hbm, out_hbm):
    look = pl.Buffered(buffer_count=2, use_lookahead=True)
    wspec = functools.partial(pl.BlockSpec, pipeline_mode=look)
    pipeline = pltpu.emit_pipeline(
        functools.partial(_gffn_inner, va_hbm=va_ref),
        grid=(NT,),
        in_specs=[
            pl.BlockSpec((T, D), lambda t: (ct_ref[0, t], 0)),
            wspec((1, DFF, D), lambda t: (te_ref[0, t], 0, 0)),
            pl.BlockSpec((1, 1, DFF), lambda t: (te_ref[0, t], 0, 0)),
            wspec((1, DFF, D), lambda t: (te_ref[0, t], 0, 0)),
            pl.BlockSpec((1, 1, DFF), lambda t: (te_ref[0, t], 0, 0)),
            wspec((1, D, DFF), lambda t: (te_ref[0, t], 0, 0)),
            pl.BlockSpec((1, 1, D), lambda t: (te_ref[0, t], 0, 0)),
            pl.BlockSpec((T, 1), lambda t: (ct_ref[0, t], 0)),
        ],
        out_specs=[pl.BlockSpec((T, D), lambda t: (ct_ref[0, t], 0))],
        _explicit_indices=True,
    )
    pipeline(xs_hbm, w1_hbm, b1_hbm, w3_hbm, b3_hbm, w2_hbm, b2_hbm,
             ews_hbm, out_hbm)


def _grouped_ffn(te, va, ct, xs, W1, b1, W3, b3, W2, b2, ews2d):
    spec_smem = pl.BlockSpec(memory_space=pltpu.SMEM)
    spec_any = pl.BlockSpec(memory_space=pl.ANY)
    return pl.pallas_call(
        _gffn_outer,
        in_specs=[spec_smem, spec_smem, spec_smem] + [spec_any] * 8,
        out_specs=spec_any,
        out_shape=jax.ShapeDtypeStruct((R, D), _f32),
    )(te, va, ct, xs, W1, b1, W3, b3, W2, b2, ews2d)


def _shared_body(x_ref, w1_ref, b1_ref, w3_ref, b3_ref, w2_ref, b2_ref, out_ref):
    xb = x_ref[...]
    h1 = lax.dot_general(xb, w1_ref[...], (((1,), (1,)), ((), ())),
                         preferred_element_type=_f32) + b1_ref[...]
    h3 = lax.dot_general(xb, w3_ref[...], (((1,), (1,)), ((), ())),
                         preferred_element_type=_f32) + b3_ref[...]
    z = h1 * h3
    h = z / (1.0 + jnp.exp(-z))
    out_ref[...] = lax.dot_general(h, w2_ref[...], (((1,), (1,)), ((), ())),
                                   preferred_element_type=_f32) + b2_ref[...]


def _shared_ffn(x2, Ws1, bs1, Ws3, bs3, Ws2, bs2):
    nst = N // T
    return pl.pallas_call(
        _shared_body,
        grid=(nst,),
        in_specs=[
            pl.BlockSpec((T, D), lambda t: (t, 0)),
            pl.BlockSpec((DFF, D), lambda t: (0, 0)),
            pl.BlockSpec((1, DFF), lambda t: (0, 0)),
            pl.BlockSpec((DFF, D), lambda t: (0, 0)),
            pl.BlockSpec((1, DFF), lambda t: (0, 0)),
            pl.BlockSpec((D, DFF), lambda t: (0, 0)),
            pl.BlockSpec((1, D), lambda t: (0, 0)),
        ],
        out_specs=pl.BlockSpec((T, D), lambda t: (t, 0)),
        out_shape=jax.ShapeDtypeStruct((N, D), _f32),
    )(x2, Ws1, bs1, Ws3, bs3, Ws2, bs2)


def _combine(sh, ys, pos):
    return pl.kernel(
        _combine_body,
        out_type=jax.ShapeDtypeStruct((N, D), _f32),
        mesh=_sc_mesh(),
        scratch_types=[
            pltpu.VMEM((CH,), _i32),
            pltpu.VMEM((CH,), _i32),
            pltpu.VMEM((CH, D), _f32),
            pltpu.VMEM((CH, D), _f32),
            pltpu.VMEM((CH, D), _f32),
            pltpu.SemaphoreType.DMA,
            pltpu.SemaphoreType.DMA,
            pltpu.SemaphoreType.DMA,
        ],
    )(sh, ys, pos)


def _combine_body(sh_hbm, ys_hbm, pos_hbm, y_hbm,
                  idx0_v, idx1_v, g0_v, g1_v, shb_v, sem0, sem1, sem2):
    wid = lax.axis_index("s") * NC + lax.axis_index("c")
    base = wid * TOK_W
    for c in range(TOK_W // CH):
        b = base + c * CH
        pltpu.sync_copy(pos_hbm.at[0, pl.ds(b, CH)], idx0_v)
        pltpu.sync_copy(pos_hbm.at[1, pl.ds(b, CH)], idx1_v)
        a0 = pltpu.async_copy(ys_hbm.at[idx0_v], g0_v, sem0)
        a1 = pltpu.async_copy(ys_hbm.at[idx1_v], g1_v, sem1)
        a2 = pltpu.async_copy(sh_hbm.at[pl.ds(b, CH)], shb_v, sem2)
        a0.wait()
        a1.wait()
        a2.wait()

        def body(i, carry):
            j = i // (D // 16)
            k = (i % (D // 16)) * 16
            shb_v[j, pl.ds(k, 16)] = (shb_v[j, pl.ds(k, 16)]
                                      + g0_v[j, pl.ds(k, 16)]
                                      + g1_v[j, pl.ds(k, 16)])
            return carry

        lax.fori_loop(0, CH * (D // 16), body, 0)
        pltpu.sync_copy(shb_v, y_hbm.at[pl.ds(b, CH)])


def kernel(x, gate_w, W1, b1, W2, b2, W3, b3, Ws1, bs1, Ws2, bs2, Ws3, bs3):
    shape = x.shape
    x2 = x.reshape(-1, shape[-1])

    pos, ewp, te, va, ct = _router(x2, gate_w)

    xs, ews = _dispatch(x2, pos, ewp)
    ys = _grouped_ffn(te, va, ct, xs, W1, b1.reshape(E, 1, DFF), W3,
                      b3.reshape(E, 1, DFF), W2, b2.reshape(E, 1, D),
                      ews.reshape(R, 1))
    sh = _shared_ffn(x2, Ws1, bs1.reshape(1, DFF), Ws3, bs3.reshape(1, DFF),
                     Ws2, bs2.reshape(1, D))
    y = _combine(sh, ys, pos)
    return y.reshape(shape)


# speedup vs baseline: 2.2637x; 1.1289x over previous
import functools

import jax
import jax.numpy as jnp
from jax import lax
from jax.experimental import pallas as pl
from jax.experimental.pallas import tpu as pltpu
from jax.experimental.pallas import tpu_sc as plsc

N = 2048
D = 1024
DFF = 2048
E = 8
TOPK = 2
T = 256
NT = 24
R = NT * T
NTPAD = 128
PAIRS = N * TOPK

NC = 2
NS = 16
NW = NC * NS
TOK_W = N // NW
CH = 32

_f32 = jnp.float32
_i32 = jnp.int32


def _router_body(x_ref, gw_ref, pos_ref, ewp_ref, te_ref, va_ref, ct_ref,
                 chg_ref, sl_ref, nx_ref, hn_ref, b_scr, exc_scr):
    xv = x_ref[...]
    gw = gw_ref[...]
    logits = lax.dot_general(gw, xv, (((1,), (1,)), ((), ())),
                             preferred_element_type=_f32)
    m = jnp.max(logits, axis=0, keepdims=True)
    p = jnp.exp(logits - m)
    scores = p / jnp.sum(p, axis=0, keepdims=True)

    e_ids = lax.broadcasted_iota(_i32, (E, N), 0)
    s0 = jnp.max(scores, axis=0, keepdims=True)
    i0 = jnp.min(jnp.where(scores == s0, e_ids, E), axis=0, keepdims=True)
    mask0 = e_ids == i0
    scores1 = jnp.where(mask0, -jnp.inf, scores)
    s1 = jnp.max(scores1, axis=0, keepdims=True)
    i1 = jnp.min(jnp.where(scores1 == s1, e_ids, E), axis=0, keepdims=True)
    mask1 = e_ids == i1

    r = jnp.exp(s1 - s0)
    ew0 = 1.0 / (1.0 + r)
    ew1 = r / (1.0 + r)

    b_scr[...] = mask0.astype(_f32) + mask1.astype(_f32)

    cb = 256
    iu = lax.broadcasted_iota(_i32, (cb, cb), 0)
    ju = lax.broadcasted_iota(_i32, (cb, cb), 1)
    upper = (iu < ju).astype(_f32)

    def chunk(c, tot):
        bc = b_scr[:, pl.ds(c * cb, cb)]
        exc = lax.dot_general(bc, upper, (((1,), (0,)), ((), ())),
                              preferred_element_type=_f32) + tot
        exc_scr[:, pl.ds(c * cb, cb)] = exc
        return tot + jnp.sum(bc, axis=1, keepdims=True)

    counts = lax.fori_loop(0, N // cb, chunk, jnp.zeros((E, 1), _f32))

    pc = jnp.ceil(counts / float(T)) * float(T)
    il = lax.broadcasted_iota(_i32, (E, E), 0)
    jl = lax.broadcasted_iota(_i32, (E, E), 1)
    lower = (il > jl).astype(_f32)
    ps = lax.dot_general(lower, pc, (((1,), (0,)), ((), ())),
                         preferred_element_type=_f32)

    posall = exc_scr[...] + ps
    pos0 = jnp.sum(jnp.where(mask0, posall, 0.0), axis=0, keepdims=True)
    pos1 = jnp.sum(jnp.where(mask1, posall, 0.0), axis=0, keepdims=True)
    pos_ref[0:1, :] = pos0.astype(_i32)
    pos_ref[1:2, :] = pos1.astype(_i32)
    ewp_ref[0:1, :] = ew0
    ewp_ref[1:2, :] = ew1

    tt = (lax.broadcasted_iota(_i32, (1, NTPAD), 1) * T).astype(_f32)
    ge = (tt >= ps).astype(_i32)
    te = jnp.sum(ge, axis=0, keepdims=True) - 1
    te = jnp.clip(te, 0, E - 1)
    inseg = jnp.logical_and(tt >= ps, tt < ps + counts)
    va = jnp.sum(inseg.astype(_i32), axis=0, keepdims=True)
    e_col = lax.broadcasted_iota(_i32, (E, 1), 0)
    last_e = jnp.max(jnp.where(counts > 0, e_col, -1), axis=0, keepdims=True)
    nvalid = jnp.sum(va, axis=1, keepdims=True)
    tids = lax.broadcasted_iota(_i32, (1, NTPAD), 1)
    te = jnp.where(va != 0, te, last_e)
    te_ref[...] = te
    va_ref[...] = va
    ct_ref[...] = jnp.minimum(tids, nvalid - 1)

    te_f = te.astype(_f32)
    te_prev = jnp.concatenate([te[:, :1] - 1, te[:, :-1]], axis=1)
    chg = (te != te_prev).astype(_f32)
    iq = lax.broadcasted_iota(_i32, (NTPAD, NTPAD), 0)
    jq = lax.broadcasted_iota(_i32, (NTPAD, NTPAD), 1)
    uincl = (iq <= jq).astype(_f32)
    cum = lax.dot_general(chg, uincl, (((1,), (0,)), ((), ())),
                          preferred_element_type=_f32)
    r_id = cum - 1.0
    eye = (iq == jq).astype(_f32)
    te_col = lax.dot_general(eye, te_f, (((1,), (1,)), ((), ())),
                             preferred_element_type=_f32)
    chg_col = lax.dot_general(eye, chg, (((1,), (1,)), ((), ())),
                              preferred_element_type=_f32)
    big = jnp.float32(1e9)
    amask = jnp.logical_and(iq > jq, chg_col != 0.0)
    nxa = jnp.min(jnp.where(amask, te_col, big), axis=0, keepdims=True)
    hn = (nxa < big).astype(_i32)
    chg_ref[...] = chg.astype(_i32)
    sl_ref[...] = (r_id.astype(_i32)) % 2
    nx_ref[...] = jnp.where(hn != 0, nxa, 0.0).astype(_i32)
    hn_ref[...] = hn


def _router(x2, gate_w):
    return pl.pallas_call(
        _router_body,
        out_shape=(
            jax.ShapeDtypeStruct((TOPK, N), _i32),
            jax.ShapeDtypeStruct((TOPK, N), _f32),
            jax.ShapeDtypeStruct((1, NTPAD), _i32),
            jax.ShapeDtypeStruct((1, NTPAD), _i32),
            jax.ShapeDtypeStruct((1, NTPAD), _i32),
            jax.ShapeDtypeStruct((1, NTPAD), _i32),
            jax.ShapeDtypeStruct((1, NTPAD), _i32),
            jax.ShapeDtypeStruct((1, NTPAD), _i32),
            jax.ShapeDtypeStruct((1, NTPAD), _i32),
        ),
        scratch_shapes=[
            pltpu.VMEM((E, N), _f32),
            pltpu.VMEM((E, N), _f32),
        ],
    )(x2, gate_w)


def _sc_mesh():
    return plsc.VectorSubcoreMesh(core_axis_name="c", subcore_axis_name="s",
                                  num_cores=NC, num_subcores=NS)


def _dispatch(x2, pos, ewp):
    return pl.kernel(
        _dispatch_body,
        out_type=(
            jax.ShapeDtypeStruct((R, D), _f32),
            jax.ShapeDtypeStruct((R,), _f32),
        ),
        mesh=_sc_mesh(),
        scratch_types=[
            pltpu.VMEM((TOK_W,), _i32),
            pltpu.VMEM((TOK_W,), _i32),
            pltpu.VMEM((TOK_W,), _f32),
            pltpu.VMEM((TOK_W,), _f32),
            pltpu.VMEM((TOK_W, D), _f32),
            pltpu.SemaphoreType.DMA,
            pltpu.SemaphoreType.DMA,
            pltpu.SemaphoreType.DMA,
            pltpu.SemaphoreType.DMA,
        ],
    )(x2, pos, ewp)


def _dispatch_body(x_hbm, pos_hbm, ewp_hbm, xs_hbm, ews_hbm,
                   idx0_v, idx1_v, ew0_v, ew1_v, rows_v, sem0, sem1, sem2, sem3):
    wid = lax.axis_index("s") * NC + lax.axis_index("c")
    base = wid * TOK_W
    pltpu.sync_copy(x_hbm.at[pl.ds(base, TOK_W)], rows_v)
    pltpu.sync_copy(pos_hbm.at[0, pl.ds(base, TOK_W)], idx0_v)
    pltpu.sync_copy(pos_hbm.at[1, pl.ds(base, TOK_W)], idx1_v)
    pltpu.sync_copy(ewp_hbm.at[0, pl.ds(base, TOK_W)], ew0_v)
    pltpu.sync_copy(ewp_hbm.at[1, pl.ds(base, TOK_W)], ew1_v)
    cp0 = pltpu.async_copy(rows_v, xs_hbm.at[idx0_v], sem0)
    cp1 = pltpu.async_copy(rows_v, xs_hbm.at[idx1_v], sem1)
    cp2 = pltpu.async_copy(ew0_v, ews_hbm.at[idx0_v], sem2)
    cp3 = pltpu.async_copy(ew1_v, ews_hbm.at[idx1_v], sem3)
    cp0.wait()
    cp1.wait()
    cp2.wait()
    cp3.wait()


def _w_copies(w1_hbm, w3_hbm, w2_hbm, w1s, w3s, w2s, sem, e, slot):
    return (
        pltpu.make_async_copy(w1_hbm.at[e], w1s.at[slot], sem),
        pltpu.make_async_copy(w3_hbm.at[e], w3s.at[slot], sem),
        pltpu.make_async_copy(w2_hbm.at[e], w2s.at[slot], sem),
    )


def _gffn_inner(idx, xs_ref, ews_ref, out_ref, *, meta, hbm, scr):
    (t,) = idx
    te_ref, va_ref, chg_ref, sl_ref, nx_ref, hn_ref = meta
    w1_hbm, w3_hbm, w2_hbm = hbm
    w1s, w3s, w2s, b1s, b3s, b2s, sems = scr
    slot = sl_ref[0, t]

    @pl.when(chg_ref[0, t] != 0)
    def _():
        e_cur = te_ref[0, t]
        for cp in _w_copies(w1_hbm, w3_hbm, w2_hbm, w1s, w3s, w2s,
                            sems.at[slot], e_cur, slot):
            cp.wait()

        @pl.when(hn_ref[0, t] != 0)
        def _():
            e_nx = nx_ref[0, t]
            for cp in _w_copies(w1_hbm, w3_hbm, w2_hbm, w1s, w3s, w2s,
                                sems.at[1 - slot], e_nx, 1 - slot):
                cp.start()

    @pl.when(va_ref[0, t] != 0)
    def _():
        xb = xs_ref[...]
        h1 = lax.dot_general(xb, w1s[slot], (((1,), (1,)), ((), ())),
                             preferred_element_type=_f32) + b1s[te_ref[0, t]]
        h3 = lax.dot_general(xb, w3s[slot], (((1,), (1,)), ((), ())),
                             preferred_element_type=_f32) + b3s[te_ref[0, t]]
        z = h1 * h3
        h = z / (1.0 + jnp.exp(-z))
        acc = lax.dot_general(h, w2s[slot], (((1,), (1,)), ((), ())),
                              preferred_element_type=_f32) + b2s[te_ref[0, t]]
        out_ref[...] = acc * ews_ref[...]


def _gffn_outer(te_ref, va_ref, ct_ref, chg_ref, sl_ref, nx_ref, hn_ref,
                xs_hbm, w1_hbm, b1_ref, w3_hbm, b3_ref, w2_hbm, b2_ref,
                ews_hbm, out_hbm, w1s, w3s, w2s, sems):
    e0 = te_ref[0, 0]
    for cp in _w_copies(w1_hbm, w3_hbm, w2_hbm, w1s, w3s, w2s, sems.at[0],
                        e0, 0):
        cp.start()

    pipeline = pltpu.emit_pipeline(
        functools.partial(
            _gffn_inner,
            meta=(te_ref, va_ref, chg_ref, sl_ref, nx_ref, hn_ref),
            hbm=(w1_hbm, w3_hbm, w2_hbm),
            scr=(w1s, w3s, w2s, b1_ref, b3_ref, b2_ref, sems)),
        grid=(NT,),
        in_specs=[
            pl.BlockSpec((T, D), lambda t: (ct_ref[0, t], 0)),
            pl.BlockSpec((T, 1), lambda t: (ct_ref[0, t], 0)),
        ],
        out_specs=[pl.BlockSpec((T, D), lambda t: (ct_ref[0, t], 0))],
        _explicit_indices=True,
    )
    pipeline(xs_hbm, ews_hbm, out_hbm)


def _grouped_ffn(te, va, ct, chg, sl, nx, hn, xs, W1, b1, W3, b3, W2, b2,
                 ews2d):
    spec_smem = pl.BlockSpec(memory_space=pltpu.SMEM)
    spec_any = pl.BlockSpec(memory_space=pl.ANY)
    spec_vmem = pl.BlockSpec(memory_space=pltpu.VMEM)
    return pl.pallas_call(
        _gffn_outer,
        in_specs=[spec_smem] * 7 + [spec_any, spec_any, spec_vmem, spec_any,
                                    spec_vmem, spec_any, spec_vmem, spec_any],
        out_specs=spec_any,
        out_shape=jax.ShapeDtypeStruct((R, D), _f32),
        scratch_shapes=[
            pltpu.VMEM((2, DFF, D), _f32),
            pltpu.VMEM((2, DFF, D), _f32),
            pltpu.VMEM((2, D, DFF), _f32),
            pltpu.SemaphoreType.DMA((2,)),
        ],
    )(te, va, ct, chg, sl, nx, hn, xs, W1, b1, W3, b3, W2, b2, ews2d)


def _shared_body(x_ref, w1_ref, b1_ref, w3_ref, b3_ref, w2_ref, b2_ref, out_ref):
    xb = x_ref[...]
    h1 = lax.dot_general(xb, w1_ref[...], (((1,), (1,)), ((), ())),
                         preferred_element_type=_f32) + b1_ref[...]
    h3 = lax.dot_general(xb, w3_ref[...], (((1,), (1,)), ((), ())),
                         preferred_element_type=_f32) + b3_ref[...]
    z = h1 * h3
    h = z / (1.0 + jnp.exp(-z))
    out_ref[...] = lax.dot_general(h, w2_ref[...], (((1,), (1,)), ((), ())),
                                   preferred_element_type=_f32) + b2_ref[...]


def _shared_ffn(x2, Ws1, bs1, Ws3, bs3, Ws2, bs2):
    nst = N // T
    return pl.pallas_call(
        _shared_body,
        grid=(nst,),
        in_specs=[
            pl.BlockSpec((T, D), lambda t: (t, 0)),
            pl.BlockSpec((DFF, D), lambda t: (0, 0)),
            pl.BlockSpec((1, DFF), lambda t: (0, 0)),
            pl.BlockSpec((DFF, D), lambda t: (0, 0)),
            pl.BlockSpec((1, DFF), lambda t: (0, 0)),
            pl.BlockSpec((D, DFF), lambda t: (0, 0)),
            pl.BlockSpec((1, D), lambda t: (0, 0)),
        ],
        out_specs=pl.BlockSpec((T, D), lambda t: (t, 0)),
        out_shape=jax.ShapeDtypeStruct((N, D), _f32),
    )(x2, Ws1, bs1, Ws3, bs3, Ws2, bs2)


def _combine(sh, ys, pos):
    return pl.kernel(
        _combine_body,
        out_type=jax.ShapeDtypeStruct((N, D), _f32),
        mesh=_sc_mesh(),
        scratch_types=[
            pltpu.VMEM((CH,), _i32),
            pltpu.VMEM((CH,), _i32),
            pltpu.VMEM((CH, D), _f32),
            pltpu.VMEM((CH, D), _f32),
            pltpu.VMEM((CH, D), _f32),
            pltpu.SemaphoreType.DMA,
            pltpu.SemaphoreType.DMA,
            pltpu.SemaphoreType.DMA,
        ],
    )(sh, ys, pos)


def _combine_body(sh_hbm, ys_hbm, pos_hbm, y_hbm,
                  idx0_v, idx1_v, g0_v, g1_v, shb_v, sem0, sem1, sem2):
    wid = lax.axis_index("s") * NC + lax.axis_index("c")
    base = wid * TOK_W
    for c in range(TOK_W // CH):
        b = base + c * CH
        pltpu.sync_copy(pos_hbm.at[0, pl.ds(b, CH)], idx0_v)
        pltpu.sync_copy(pos_hbm.at[1, pl.ds(b, CH)], idx1_v)
        a0 = pltpu.async_copy(ys_hbm.at[idx0_v], g0_v, sem0)
        a1 = pltpu.async_copy(ys_hbm.at[idx1_v], g1_v, sem1)
        a2 = pltpu.async_copy(sh_hbm.at[pl.ds(b, CH)], shb_v, sem2)
        a0.wait()
        a1.wait()
        a2.wait()

        def body(i, carry):
            j = i // (D // 16)
            k = (i % (D // 16)) * 16
            shb_v[j, pl.ds(k, 16)] = (shb_v[j, pl.ds(k, 16)]
                                      + g0_v[j, pl.ds(k, 16)]
                                      + g1_v[j, pl.ds(k, 16)])
            return carry

        lax.fori_loop(0, CH * (D // 16), body, 0)
        pltpu.sync_copy(shb_v, y_hbm.at[pl.ds(b, CH)])


def kernel(x, gate_w, W1, b1, W2, b2, W3, b3, Ws1, bs1, Ws2, bs2, Ws3, bs3):
    shape = x.shape
    x2 = x.reshape(-1, shape[-1])

    pos, ewp, te, va, ct, chg, sl, nx, hn = _router(x2, gate_w)

    xs, ews = _dispatch(x2, pos, ewp)
    ys = _grouped_ffn(te, va, ct, chg, sl, nx, hn, xs, W1,
                      b1.reshape(E, 1, DFF), W3, b3.reshape(E, 1, DFF), W2,
                      b2.reshape(E, 1, D), ews.reshape(R, 1))
    sh = _shared_ffn(x2, Ws1, bs1.reshape(1, DFF), Ws3, bs3.reshape(1, DFF),
                     Ws2, bs2.reshape(1, D))
    y = _combine(sh, ys, pos)
    return y.reshape(shape)
